# Initial kernel scaffold; baseline (speedup 1.0000x reference)
import jax
import jax.numpy as jnp
from jax.experimental import pallas as pl


def kernel(A, h, Wq, bq, Wk, bk, Wv, bv):
    raise NotImplementedError("write your pallas kernel here")


# speedup vs baseline: 38.1284x; 38.1284x over previous
import dataclasses
import functools

import jax
import jax.numpy as jnp
from jax import lax
from jax.experimental import pallas as pl
from jax.experimental.pallas import tpu as pltpu
from jax.experimental.pallas import tpu_sc as plsc

N = 10000
E = 320000
HID = 128
NH = 8
HD = 16
NC = 2
NS = 16
NW = NC * NS
EPW = E // NW
BLK = 80
NBLK = EPW // BLK
NP = 10240
ND = NP // 16
RPS = NP // NS
DPS = ND // NS
BN = 1024


def _qkv(h, wq, bq, wk, bk, wv, bv):
    dn = (((1,), (1,)), ((), ()))
    scale = HD ** -0.5

    def body(h_ref, wq_ref, bq_ref, wk_ref, bk_ref, wv_ref, bv_ref,
             q_ref, k_ref, v_ref):
        hb = h_ref[...]
        q_ref[...] = (lax.dot_general(hb, wq_ref[...], dn,
                                      preferred_element_type=jnp.float32)
                      + bq_ref[...]) * scale
        k_ref[...] = lax.dot_general(hb, wk_ref[...], dn,
                                     preferred_element_type=jnp.float32) + bk_ref[...]
        v_ref[...] = lax.dot_general(hb, wv_ref[...], dn,
                                     preferred_element_type=jnp.float32) + bv_ref[...]

    wspec = pl.BlockSpec((HID, HID), lambda i: (0, 0))
    bspec = pl.BlockSpec((1, HID), lambda i: (0, 0))
    out = jax.ShapeDtypeStruct((N, HID), jnp.float32)
    return pl.pallas_call(
        body,
        grid=(N // 1000,),
        in_specs=[pl.BlockSpec((1000, HID), lambda i: (i, 0)),
                  wspec, bspec, wspec, bspec, wspec, bspec],
        out_specs=[pl.BlockSpec((1000, HID), lambda i: (i, 0))] * 3,
        out_shape=[out, out, out],
    )(h, wq, bq.reshape(1, HID), wk, bk.reshape(1, HID),
      wv, bv.reshape(1, HID))


def _edge_sc(q, k, v, row, col, zeros):
    mesh = plsc.VectorSubcoreMesh(core_axis_name="c", subcore_axis_name="s")
    cp = pltpu.CompilerParams()
    if "needs_layout_passes" in pltpu.CompilerParams.__dataclass_fields__:
        cp = dataclasses.replace(cp, needs_layout_passes=False)
    if "use_tc_tiling_on_sc" in pltpu.CompilerParams.__dataclass_fields__:
        cp = dataclasses.replace(cp, use_tc_tiling_on_sc=False)

    @functools.partial(
        pl.kernel,
        out_type=[jax.ShapeDtypeStruct((NC, NP, HID), jnp.float32),
                  jax.ShapeDtypeStruct((NC, ND, HID), jnp.float32)],
        mesh=mesh,
        compiler_params=cp,
        scratch_types=[
            pltpu.VMEM((BLK,), jnp.int32),
            pltpu.VMEM((BLK,), jnp.int32),
            pltpu.VMEM((BLK,), jnp.int32),
            pltpu.VMEM((BLK,), jnp.int32),
            pltpu.VMEM((BLK, HID), jnp.float32),
            pltpu.VMEM((BLK, HID), jnp.float32),
            pltpu.VMEM((BLK, HID), jnp.float32),
            pltpu.VMEM((BLK, HID), jnp.float32),
            pltpu.VMEM((16,), jnp.float32),
            pltpu.VMEM_SHARED((NP, HID), jnp.float32),
            pltpu.VMEM_SHARED((ND, HID), jnp.float32),
        ],
    )
    def k_(q_hbm, k_hbm, v_hbm, row_hbm, col_hbm, z_hbm, out_hbm, outd_hbm,
           ridx, cidx, didx, poff, qg, kg, vg, dmsg, fold, acc, accD):
        cid = lax.axis_index("c")
        sid = lax.axis_index("s")
        wid = sid * NC + cid
        r0 = pl.multiple_of(sid * RPS, 8)
        d0 = pl.multiple_of(sid * DPS, 8)
        pltpu.sync_copy(z_hbm.at[cid].at[pl.ds(r0, RPS)],
                        acc.at[pl.ds(r0, RPS)])
        pltpu.sync_copy(z_hbm.at[cid].at[pl.ds(d0, DPS)],
                        accD.at[pl.ds(d0, DPS)])

        iota = lax.broadcasted_iota(jnp.int32, (16,), 0)
        perm = iota ^ 8
        lo8 = iota < 8
        z16 = jnp.zeros((16,), jnp.float32)
        @pl.loop(0, BLK)
        def _(e):
            for j in range(8):
                dmsg[e, pl.ds(16 * j, 16)] = z16

        plsc.subcore_barrier()
        e0 = wid * EPW

        @pl.loop(0, NBLK)
        def _(b):
            base = pl.multiple_of(e0 + b * BLK, 8)
            pltpu.sync_copy(row_hbm.at[pl.ds(base, BLK)], ridx)
            pltpu.sync_copy(col_hbm.at[pl.ds(base, BLK)], cidx)
            pltpu.sync_copy(q_hbm.at[ridx], qg)
            pltpu.sync_copy(k_hbm.at[cidx], kg)
            pltpu.sync_copy(v_hbm.at[cidx], vg)

            @pl.loop(0, BLK, step=16)
            def _(c):
                r = ridx[pl.ds(c, 16)]
                didx[pl.ds(c, 16)] = jnp.right_shift(r, 4)
                poff[pl.ds(c, 16)] = jnp.bitwise_and(r, 15) * 8

            @pl.loop(0, BLK)
            def _(e):
                t = qg[e, pl.ds(0, 16)] * kg[e, pl.ds(0, 16)]
                for j in range(1, 8):
                    t = t + qg[e, pl.ds(16 * j, 16)] * kg[e, pl.ds(16 * j, 16)]
                fold[...] = t
                m = jnp.exp(t + plsc.load_gather(fold, [perm]))
                for j in range(8):
                    vg[e, pl.ds(16 * j, 16)] = m * vg[e, pl.ds(16 * j, 16)]
                esplat = jnp.full((16,), e, jnp.int32)
                lanes = plsc.load_gather(poff, [esplat]) + iota
                plsc.store_scatter(dmsg, [esplat, lanes], m, mask=lo8)

            pltpu.sync_copy(vg, acc.at[ridx], add=True)
            pltpu.sync_copy(dmsg, accD.at[didx], add=True)

            @pl.loop(0, BLK)
            def _(e):
                esplat = jnp.full((16,), e, jnp.int32)
                lanes = plsc.load_gather(poff, [esplat]) + iota
                plsc.store_scatter(dmsg, [esplat, lanes], z16, mask=lo8)

        plsc.subcore_barrier()
        pltpu.sync_copy(acc.at[pl.ds(r0, RPS)],
                        out_hbm.at[cid].at[pl.ds(r0, RPS)])
        pltpu.sync_copy(accD.at[pl.ds(d0, DPS)],
                        outd_hbm.at[cid].at[pl.ds(d0, DPS)])

    return k_(q, k, v, row, col, zeros)


def _combine(parts, dparts):
    def body(p_ref, d_ref, o_ref):
        s = p_ref[0] + p_ref[1]
        den8 = d_ref[0] + d_ref[1]
        den16 = jnp.concatenate([den8, den8], axis=1)
        den = jnp.concatenate([den16] * (HID // 16), axis=1)
        o_ref[...] = s / jnp.where(den == 0.0, 1.0, den)

    return pl.pallas_call(
        body,
        grid=(NP // BN,),
        in_specs=[pl.BlockSpec((NC, BN, HID), lambda i: (0, i, 0)),
                  pl.BlockSpec((NC, BN, NH), lambda i: (0, i, 0))],
        out_specs=pl.BlockSpec((BN, HID), lambda i: (i, 0)),
        out_shape=jax.ShapeDtypeStruct((NP, HID), jnp.float32),
    )(parts, dparts.reshape(NC, NP, NH))


@jax.jit
def kernel(A, h, Wq, bq, Wk, bk, Wv, bv):
    row = A[0].astype(jnp.int32)
    col = A[1].astype(jnp.int32)
    q, k, v = _qkv(h, Wq, bq, Wk, bk, Wv, bv)
    zeros = jnp.zeros((NC, NP, HID), jnp.float32)
    parts, dparts = _edge_sc(q, k, v, row, col, zeros)
    return _combine(parts, dparts)[:N]


# speedup vs baseline: 46.2259x; 1.2124x over previous
import dataclasses
import functools

import jax
import jax.numpy as jnp
from jax import lax
from jax.experimental import pallas as pl
from jax.experimental.pallas import tpu as pltpu
from jax.experimental.pallas import tpu_sc as plsc

N = 10000
E = 320000
HID = 128
NH = 8
HD = 16
NC = 2
NS = 16
NW = NC * NS
EPW = E // NW
BLK = 80
NBLK = EPW // BLK
NP = 10240
RPS = NP // NS
BN = 1024


def _qkv(h, wq, bq, wk, bk, wv, bv):
    dn = (((1,), (1,)), ((), ()))
    scale = HD ** -0.5

    def body(h_ref, wq_ref, bq_ref, wk_ref, bk_ref, wv_ref, bv_ref,
             q_ref, k_ref, v_ref):
        hb = h_ref[...]
        q_ref[...] = (lax.dot_general(hb, wq_ref[...], dn,
                                      preferred_element_type=jnp.float32)
                      + bq_ref[...]) * scale
        k_ref[...] = lax.dot_general(hb, wk_ref[...], dn,
                                     preferred_element_type=jnp.float32) + bk_ref[...]
        v_ref[...] = lax.dot_general(hb, wv_ref[...], dn,
                                     preferred_element_type=jnp.float32) + bv_ref[...]

    wspec = pl.BlockSpec((HID, HID), lambda i: (0, 0))
    bspec = pl.BlockSpec((1, HID), lambda i: (0, 0))
    out = jax.ShapeDtypeStruct((N, HID), jnp.float32)
    return pl.pallas_call(
        body,
        grid=(N // 1000,),
        in_specs=[pl.BlockSpec((1000, HID), lambda i: (i, 0)),
                  wspec, bspec, wspec, bspec, wspec, bspec],
        out_specs=[pl.BlockSpec((1000, HID), lambda i: (i, 0))] * 3,
        out_shape=[out, out, out],
    )(h, wq, bq.reshape(1, HID), wk, bk.reshape(1, HID),
      wv, bv.reshape(1, HID))


def _edge_sc(q, k, v, row, col, zeros, zerosd):
    mesh = plsc.VectorSubcoreMesh(core_axis_name="c", subcore_axis_name="s")
    cp = pltpu.CompilerParams()
    if "needs_layout_passes" in pltpu.CompilerParams.__dataclass_fields__:
        cp = dataclasses.replace(cp, needs_layout_passes=False)
    if "use_tc_tiling_on_sc" in pltpu.CompilerParams.__dataclass_fields__:
        cp = dataclasses.replace(cp, use_tc_tiling_on_sc=False)

    @functools.partial(
        pl.kernel,
        out_type=[jax.ShapeDtypeStruct((NC, NP, HID), jnp.float32),
                  jax.ShapeDtypeStruct((NC, NP, 16), jnp.float32)],
        mesh=mesh,
        compiler_params=cp,
        scratch_types=[
            pltpu.VMEM((BLK,), jnp.int32),
            pltpu.VMEM((BLK,), jnp.int32),
            pltpu.VMEM((BLK, HID), jnp.float32),
            pltpu.VMEM((BLK, HID), jnp.float32),
            pltpu.VMEM((BLK, HID), jnp.float32),
            pltpu.VMEM((BLK, 16), jnp.float32),
            pltpu.VMEM((16,), jnp.float32),
            pltpu.VMEM_SHARED((NP, HID), jnp.float32),
            pltpu.VMEM_SHARED((NP, 16), jnp.float32),
        ],
    )
    def k_(q_hbm, k_hbm, v_hbm, row_hbm, col_hbm, z_hbm, zd_hbm,
           out_hbm, outd_hbm,
           ridx, cidx, qg, kg, vg, dmsg, fold, acc, accD):
        cid = lax.axis_index("c")
        sid = lax.axis_index("s")
        wid = sid * NC + cid
        r0 = pl.multiple_of(sid * RPS, 8)
        pltpu.sync_copy(z_hbm.at[cid].at[pl.ds(r0, RPS)],
                        acc.at[pl.ds(r0, RPS)])
        pltpu.sync_copy(zd_hbm.at[cid].at[pl.ds(r0, RPS)],
                        accD.at[pl.ds(r0, RPS)])

        iota = lax.broadcasted_iota(jnp.int32, (16,), 0)
        perm = iota ^ 8

        plsc.subcore_barrier()
        e0 = wid * EPW

        @pl.loop(0, NBLK)
        def _(b):
            base = pl.multiple_of(e0 + b * BLK, 8)
            pltpu.sync_copy(row_hbm.at[pl.ds(base, BLK)], ridx)
            pltpu.sync_copy(col_hbm.at[pl.ds(base, BLK)], cidx)
            pltpu.sync_copy(q_hbm.at[ridx], qg)
            pltpu.sync_copy(k_hbm.at[cidx], kg)
            pltpu.sync_copy(v_hbm.at[cidx], vg)

            @pl.loop(0, BLK)
            def _(e):
                t = qg[e, pl.ds(0, 16)] * kg[e, pl.ds(0, 16)]
                for j in range(1, 8):
                    t = t + qg[e, pl.ds(16 * j, 16)] * kg[e, pl.ds(16 * j, 16)]
                fold[...] = t
                m = jnp.exp(t + plsc.load_gather(fold, [perm]))
                for j in range(8):
                    vg[e, pl.ds(16 * j, 16)] = m * vg[e, pl.ds(16 * j, 16)]
                dmsg[e, pl.ds(0, 16)] = m

            pltpu.sync_copy(vg, acc.at[ridx], add=True)
            pltpu.sync_copy(dmsg, accD.at[ridx], add=True)

        plsc.subcore_barrier()
        pltpu.sync_copy(acc.at[pl.ds(r0, RPS)],
                        out_hbm.at[cid].at[pl.ds(r0, RPS)])
        pltpu.sync_copy(accD.at[pl.ds(r0, RPS)],
                        outd_hbm.at[cid].at[pl.ds(r0, RPS)])

    return k_(q, k, v, row, col, zeros, zerosd)


def _combine(parts, dparts):
    def body(p_ref, d_ref, o_ref):
        s = p_ref[0] + p_ref[1]
        den16 = d_ref[0] + d_ref[1]
        den = jnp.concatenate([den16] * (HID // 16), axis=1)
        o_ref[...] = s / jnp.where(den == 0.0, 1.0, den)

    return pl.pallas_call(
        body,
        grid=(NP // BN,),
        in_specs=[pl.BlockSpec((NC, BN, HID), lambda i: (0, i, 0)),
                  pl.BlockSpec((NC, BN, 16), lambda i: (0, i, 0))],
        out_specs=pl.BlockSpec((BN, HID), lambda i: (i, 0)),
        out_shape=jax.ShapeDtypeStruct((NP, HID), jnp.float32),
    )(parts, dparts)


@jax.jit
def kernel(A, h, Wq, bq, Wk, bk, Wv, bv):
    row = A[0].astype(jnp.int32)
    col = A[1].astype(jnp.int32)
    q, k, v = _qkv(h, Wq, bq, Wk, bk, Wv, bv)
    zeros = jnp.zeros((NC, NP, HID), jnp.float32)
    zerosd = jnp.zeros((NC, NP, 16), jnp.float32)
    parts, dparts = _edge_sc(q, k, v, row, col, zeros, zerosd)
    return _combine(parts, dparts)[:N]


# speedup vs baseline: 59.3056x; 1.2829x over previous
import dataclasses
import functools

import jax
import jax.numpy as jnp
from jax import lax
from jax.experimental import pallas as pl
from jax.experimental.pallas import tpu as pltpu
from jax.experimental.pallas import tpu_sc as plsc

N = 10000
E = 320000
HID = 128
NH = 8
HD = 16
NC = 2
NS = 16
NW = NC * NS
EPW = E // NW
BLK = 80
NBLK = EPW // BLK
NP = 10240
RPS = NP // NS
BN = 1024


def _qkv(h, wq, bq, wk, bk, wv, bv):
    dn = (((1,), (1,)), ((), ()))
    scale = HD ** -0.5

    def body(h_ref, wq_ref, bq_ref, wk_ref, bk_ref, wv_ref, bv_ref,
             q_ref, k_ref, v_ref):
        hb = h_ref[...]
        q_ref[...] = (lax.dot_general(hb, wq_ref[...], dn,
                                      preferred_element_type=jnp.float32)
                      + bq_ref[...]) * scale
        k_ref[...] = lax.dot_general(hb, wk_ref[...], dn,
                                     preferred_element_type=jnp.float32) + bk_ref[...]
        v_ref[...] = lax.dot_general(hb, wv_ref[...], dn,
                                     preferred_element_type=jnp.float32) + bv_ref[...]

    wspec = pl.BlockSpec((HID, HID), lambda i: (0, 0))
    bspec = pl.BlockSpec((1, HID), lambda i: (0, 0))
    out = jax.ShapeDtypeStruct((N, HID), jnp.float32)
    return pl.pallas_call(
        body,
        grid=(N // 1000,),
        in_specs=[pl.BlockSpec((1000, HID), lambda i: (i, 0)),
                  wspec, bspec, wspec, bspec, wspec, bspec],
        out_specs=[pl.BlockSpec((1000, HID), lambda i: (i, 0))] * 3,
        out_shape=[out, out, out],
    )(h, wq, bq.reshape(1, HID), wk, bk.reshape(1, HID),
      wv, bv.reshape(1, HID))


def _edge_sc(q, k, v, row, col, zeros, zerosd):
    mesh = plsc.VectorSubcoreMesh(core_axis_name="c", subcore_axis_name="s")
    cp = pltpu.CompilerParams()
    if "needs_layout_passes" in pltpu.CompilerParams.__dataclass_fields__:
        cp = dataclasses.replace(cp, needs_layout_passes=False)
    if "use_tc_tiling_on_sc" in pltpu.CompilerParams.__dataclass_fields__:
        cp = dataclasses.replace(cp, use_tc_tiling_on_sc=False)

    @functools.partial(
        pl.kernel,
        out_type=[jax.ShapeDtypeStruct((NC, NP, HID), jnp.float32),
                  jax.ShapeDtypeStruct((NC, NP, 16), jnp.float32)],
        mesh=mesh,
        compiler_params=cp,
        scratch_types=[
            pltpu.VMEM((BLK,), jnp.int32),
            pltpu.VMEM((BLK,), jnp.int32),
            pltpu.VMEM((BLK, HID), jnp.float32),
            pltpu.VMEM((BLK, HID), jnp.float32),
            pltpu.VMEM((BLK, HID), jnp.float32),
            pltpu.VMEM((BLK, 16), jnp.float32),
            pltpu.VMEM((16,), jnp.float32),
            pltpu.VMEM_SHARED((NP, HID), jnp.float32),
            pltpu.VMEM_SHARED((NP, 16), jnp.float32),
            pltpu.SemaphoreType.DMA,
        ],
    )
    def k_(q_hbm, k_hbm, v_hbm, row_hbm, col_hbm, z_hbm, zd_hbm,
           out_hbm, outd_hbm,
           ridx, cidx, qg, kg, vg, dmsg, fold, acc, accD, sem):
        cid = lax.axis_index("c")
        sid = lax.axis_index("s")
        wid = sid * NC + cid
        r0 = pl.multiple_of(sid * RPS, 8)
        pltpu.sync_copy(z_hbm.at[cid].at[pl.ds(r0, RPS)],
                        acc.at[pl.ds(r0, RPS)])
        pltpu.sync_copy(zd_hbm.at[cid].at[pl.ds(r0, RPS)],
                        accD.at[pl.ds(r0, RPS)])

        iota = lax.broadcasted_iota(jnp.int32, (16,), 0)
        perm = iota ^ 8

        plsc.subcore_barrier()
        e0 = wid * EPW

        @pl.loop(0, NBLK)
        def _(b):
            base = pl.multiple_of(e0 + b * BLK, 8)
            ri = pltpu.async_copy(row_hbm.at[pl.ds(base, BLK)], ridx, sem)
            ci = pltpu.async_copy(col_hbm.at[pl.ds(base, BLK)], cidx, sem)
            ri.wait()
            ci.wait()
            g1 = pltpu.async_copy(q_hbm.at[ridx], qg, sem)
            g2 = pltpu.async_copy(k_hbm.at[cidx], kg, sem)
            g3 = pltpu.async_copy(v_hbm.at[cidx], vg, sem)
            g1.wait()
            g2.wait()
            g3.wait()

            @pl.loop(0, BLK)
            def _(e):
                t = qg[e, pl.ds(0, 16)] * kg[e, pl.ds(0, 16)]
                for j in range(1, 8):
                    t = t + qg[e, pl.ds(16 * j, 16)] * kg[e, pl.ds(16 * j, 16)]
                fold[...] = t
                m = jnp.exp(t + plsc.load_gather(fold, [perm]))
                for j in range(8):
                    vg[e, pl.ds(16 * j, 16)] = m * vg[e, pl.ds(16 * j, 16)]
                dmsg[e, pl.ds(0, 16)] = m

            s1 = pltpu.async_copy(vg, acc.at[ridx], sem, add=True)
            s2 = pltpu.async_copy(dmsg, accD.at[ridx], sem, add=True)
            s1.wait()
            s2.wait()

        plsc.subcore_barrier()
        pltpu.sync_copy(acc.at[pl.ds(r0, RPS)],
                        out_hbm.at[cid].at[pl.ds(r0, RPS)])
        pltpu.sync_copy(accD.at[pl.ds(r0, RPS)],
                        outd_hbm.at[cid].at[pl.ds(r0, RPS)])

    return k_(q, k, v, row, col, zeros, zerosd)


def _combine(parts, dparts):
    def body(p_ref, d_ref, o_ref):
        s = p_ref[0] + p_ref[1]
        den16 = d_ref[0] + d_ref[1]
        den = jnp.concatenate([den16] * (HID // 16), axis=1)
        o_ref[...] = s / jnp.where(den == 0.0, 1.0, den)

    return pl.pallas_call(
        body,
        grid=(NP // BN,),
        in_specs=[pl.BlockSpec((NC, BN, HID), lambda i: (0, i, 0)),
                  pl.BlockSpec((NC, BN, 16), lambda i: (0, i, 0))],
        out_specs=pl.BlockSpec((BN, HID), lambda i: (i, 0)),
        out_shape=jax.ShapeDtypeStruct((NP, HID), jnp.float32),
    )(parts, dparts)


@jax.jit
def kernel(A, h, Wq, bq, Wk, bk, Wv, bv):
    row = A[0].astype(jnp.int32)
    col = A[1].astype(jnp.int32)
    q, k, v = _qkv(h, Wq, bq, Wk, bk, Wv, bv)
    zeros = jnp.zeros((NC, NP, HID), jnp.float32)
    zerosd = jnp.zeros((NC, NP, 16), jnp.float32)
    parts, dparts = _edge_sc(q, k, v, row, col, zeros, zerosd)
    return _combine(parts, dparts)[:N]
